# baseline (speedup 1.0000x reference)
import jax
import jax.numpy as jnp
from jax import lax
from jax.experimental import pallas as pl
from jax.experimental.pallas import tpu as pltpu
from jax.experimental.pallas import tpu_sc as plsc

HIDDEN = 2048
EXPERTS = 16
TOPK = 8
BLOCK = 1024
N_TOKENS = 16384

NW = 32
CHUNK = N_TOKENS // NW
NGROUPS = CHUNK // 16


def _logits_kernel(hs_ref, w_ref, out_ref):
    out_ref[...] = jax.lax.dot_general(
        w_ref[...], hs_ref[...],
        dimension_numbers=(((1,), (1,)), ((), ())),
        preferred_element_type=jnp.float32,
    )


def _tc_logits(hs, W):
    n = hs.shape[0]
    return pl.pallas_call(
        _logits_kernel,
        grid=(n // BLOCK,),
        in_specs=[
            pl.BlockSpec((BLOCK, HIDDEN), lambda i: (i, 0)),
            pl.BlockSpec((EXPERTS, HIDDEN), lambda i: (0, 0)),
        ],
        out_specs=pl.BlockSpec((EXPERTS, BLOCK), lambda i: (0, i)),
        out_shape=jax.ShapeDtypeStruct((EXPERTS, n), jnp.float32),
    )(hs, W)


def _sc_topk_body(logits_hbm, w_hbm, i_hbm, lg_v, wv, iv):
    wid = lax.axis_index("s") * 2 + lax.axis_index("c")
    base = wid * CHUNK
    for e in range(EXPERTS):
        pltpu.sync_copy(logits_hbm.at[pl.ds(e * N_TOKENS + base, CHUNK)],
                        lg_v.at[pl.ds(e * CHUNK, CHUNK)])

    def body(g, _):
        off = g * 16
        vals = [lg_v[pl.ds(e * CHUNK + off, 16)] for e in range(EXPERTS)]
        neg = jnp.full((16,), -jnp.inf, jnp.float32)
        tops = []
        idxs = []
        for _k in range(TOPK):
            m = vals[0]
            for e in range(1, EXPERTS):
                m = jnp.maximum(m, vals[e])
            idx = jnp.full((16,), EXPERTS - 1, jnp.int32)
            for e in range(EXPERTS - 1, -1, -1):
                idx = jnp.where(vals[e] == m, jnp.full((16,), e, jnp.int32), idx)
            for e in range(EXPERTS):
                vals[e] = jnp.where(idx == jnp.full((16,), e, jnp.int32),
                                    neg, vals[e])
            tops.append(m)
            idxs.append(idx)
        es = [jnp.exp(t - tops[0]) for t in tops]
        s = es[0]
        for k in range(1, TOPK):
            s = s + es[k]
        for k in range(TOPK):
            wv[pl.ds(k * CHUNK + off, 16)] = es[k] / s
            iv[pl.ds(k * CHUNK + off, 16)] = idxs[k]
        return ()

    lax.fori_loop(0, NGROUPS, body, ())
    for k in range(TOPK):
        pltpu.sync_copy(wv.at[pl.ds(k * CHUNK, CHUNK)],
                        w_hbm.at[pl.ds(k * N_TOKENS + base, CHUNK)])
        pltpu.sync_copy(iv.at[pl.ds(k * CHUNK, CHUNK)],
                        i_hbm.at[pl.ds(k * N_TOKENS + base, CHUNK)])


@jax.jit
def kernel(hidden_states, W):
    hs = hidden_states.reshape(-1, HIDDEN)
    n = hs.shape[0]
    logits = _tc_logits(hs, W).reshape(-1)

    mesh = plsc.VectorSubcoreMesh(core_axis_name="c", subcore_axis_name="s")
    sc = pl.kernel(
        _sc_topk_body,
        mesh=mesh,
        out_type=[
            jax.ShapeDtypeStruct((TOPK * n,), jnp.float32),
            jax.ShapeDtypeStruct((TOPK * n,), jnp.int32),
        ],
        scratch_types=[
            pltpu.VMEM((EXPERTS * CHUNK,), jnp.float32),
            pltpu.VMEM((TOPK * CHUNK,), jnp.float32),
            pltpu.VMEM((TOPK * CHUNK,), jnp.int32),
        ],
    )
    w_flat, i_flat = sc(logits)
    w = w_flat.reshape(TOPK, n).T
    i = i_flat.reshape(TOPK, n).T
    return (w, i)


# speedup vs baseline: 1.8505x; 1.8505x over previous
import jax
import jax.numpy as jnp
from jax.experimental import pallas as pl

HIDDEN = 2048
EXPERTS = 16
TOPK = 8
BLOCK = 1024


def _gate_kernel(hs_ref, w_ref, w_out_ref, i_out_ref):
    logits = jax.lax.dot_general(
        w_ref[...], hs_ref[...],
        dimension_numbers=(((1,), (1,)), ((), ())),
        preferred_element_type=jnp.float32,
    )
    sub = jax.lax.broadcasted_iota(jnp.int32, logits.shape, 0)
    vals = logits
    top_vals = []
    top_idx = []
    for _ in range(TOPK):
        m = jnp.max(vals, axis=0, keepdims=True)
        is_max = vals == m
        idx = jnp.min(jnp.where(is_max, sub, EXPERTS), axis=0, keepdims=True)
        top_vals.append(m)
        top_idx.append(idx)
        vals = jnp.where(sub == idx, -jnp.inf, vals)
    v = jnp.concatenate(top_vals, axis=0)
    e = jnp.exp(v - v[:1, :])
    w_out_ref[...] = e / jnp.sum(e, axis=0, keepdims=True)
    i_out_ref[...] = jnp.concatenate(top_idx, axis=0)


@jax.jit
def kernel(hidden_states, W):
    hs = hidden_states.reshape(-1, HIDDEN)
    n = hs.shape[0]
    grid = (n // BLOCK,)
    w_out, i_out = pl.pallas_call(
        _gate_kernel,
        grid=grid,
        in_specs=[
            pl.BlockSpec((BLOCK, HIDDEN), lambda i: (i, 0)),
            pl.BlockSpec((EXPERTS, HIDDEN), lambda i: (0, 0)),
        ],
        out_specs=[
            pl.BlockSpec((TOPK, BLOCK), lambda i: (0, i)),
            pl.BlockSpec((TOPK, BLOCK), lambda i: (0, i)),
        ],
        out_shape=[
            jax.ShapeDtypeStruct((TOPK, n), jnp.float32),
            jax.ShapeDtypeStruct((TOPK, n), jnp.int32),
        ],
    )(hs, W)
    return (w_out.T, i_out.T)
